# baseline (speedup 1.0000x reference)
import jax
import jax.numpy as jnp
from jax.experimental import pallas as pl

_N = 512
_L = 24
_B = 64
_C = 24
_TB = 64


def _prologue_body(adj_ref, ws1_ref, wn1_ref, ws2_ref, wn2_ref, b1c_ref,
                   b2c_ref, at_ref, a2t_ref, adj16_ref, bv_ref, mcat_ref):
    adj = adj_ref[...]
    mask = (adj != 0.0).astype(jnp.float32)
    deg = jnp.sum(mask, axis=0, keepdims=True)
    inv = jnp.where(deg > 0.0, 1.0 / jnp.maximum(deg, 1.0), 0.0)
    at = mask * inv
    at16 = at.astype(jnp.bfloat16)
    at_ref[...] = at16
    a2t_ref[...] = jnp.dot(at16, at16,
                           preferred_element_type=jnp.float32).astype(jnp.bfloat16)
    adj16_ref[...] = adj.astype(jnp.bfloat16)
    ws1 = ws1_ref[...]
    wn1 = wn1_ref[...]
    ws2 = ws2_ref[...]
    wn2 = wn2_ref[...]
    mcat = jnp.concatenate([
        jnp.dot(ws2, ws1, preferred_element_type=jnp.float32),
        (jnp.dot(ws2, wn1, preferred_element_type=jnp.float32)
         + jnp.dot(wn2, ws1, preferred_element_type=jnp.float32)),
        jnp.dot(wn2, wn1, preferred_element_type=jnp.float32),
    ], axis=0)
    mcat_ref[...] = mcat.astype(jnp.bfloat16)
    b1c = b1c_ref[...]
    bconst = jnp.dot(ws2, b1c, preferred_element_type=jnp.float32) + b2c_ref[...]
    bneigh = jnp.dot(wn2, b1c, preferred_element_type=jnp.float32)
    ind = (deg > 0.0).astype(jnp.float32)
    bv_ref[...] = (jnp.tile(bconst, (_B, 1))
                   + jnp.tile(bneigh, (_B, 1)) * ind)


def _main_body(x_ref, at_ref, a2t_ref, adj_ref, bv_ref, mcat_ref, out_ref):
    xv = x_ref[...].reshape(_TB, _L, _N)
    mcat_b = jnp.broadcast_to(mcat_ref[...][None], (_TB, 3 * _L, _L))
    xall = jax.lax.dot_general(
        mcat_b, xv, (((2,), (1,)), ((0,), (0,))),
        preferred_element_type=jnp.float32)
    x1 = xall[:, 0:_L, :].reshape(_TB * _L, _N)
    x2 = xall[:, _L:2 * _L, :].reshape(_TB * _L, _N).astype(jnp.bfloat16)
    x3 = xall[:, 2 * _L:3 * _L, :].reshape(_TB * _L, _N).astype(jnp.bfloat16)
    res = (x1
           + jnp.dot(x2, at_ref[...], preferred_element_type=jnp.float32)
           + jnp.dot(x3, a2t_ref[...], preferred_element_type=jnp.float32)
           + bv_ref[...])
    z = jnp.dot(res.astype(jnp.bfloat16), adj_ref[...],
                preferred_element_type=jnp.float32)
    out_ref[:, 0, :, :] = (3.5 * res).reshape(_TB, _L, _N)
    out_ref[:, 1, :, :] = z.reshape(_TB, _L, _N)


@jax.jit
def kernel(x, adj, W_self1, W_neigh1, b1, W_self2, W_neigh2, b2):
    B, C, N, L = x.shape

    at, a2t, adj16, bv, mcat = pl.pallas_call(
        _prologue_body,
        out_shape=(
            jax.ShapeDtypeStruct((N, N), jnp.bfloat16),
            jax.ShapeDtypeStruct((N, N), jnp.bfloat16),
            jax.ShapeDtypeStruct((N, N), jnp.bfloat16),
            jax.ShapeDtypeStruct((B * L, N), jnp.float32),
            jax.ShapeDtypeStruct((3 * L, L), jnp.bfloat16),
        ),
    )(adj, W_self1, W_neigh1, W_self2, W_neigh2, b1[:, None], b2[:, None])

    xt = jnp.swapaxes(x, 2, 3).astype(jnp.bfloat16)

    outt = pl.pallas_call(
        _main_body,
        grid=(C,),
        in_specs=[
            pl.BlockSpec((_TB, 1, L, N), lambda c: (0, c, 0, 0)),
            pl.BlockSpec((N, N), lambda c: (0, 0)),
            pl.BlockSpec((N, N), lambda c: (0, 0)),
            pl.BlockSpec((N, N), lambda c: (0, 0)),
            pl.BlockSpec((_TB * L, N), lambda c: (0, 0)),
            pl.BlockSpec((3 * L, L), lambda c: (0, 0)),
        ],
        out_specs=pl.BlockSpec((_TB, 2, L, N), lambda c: (0, c, 0, 0)),
        out_shape=jax.ShapeDtypeStruct((B, 2 * C, L, N), jnp.float32),
    )(xt, at, a2t, adj16, bv, mcat)

    return jnp.swapaxes(outt, 2, 3)


# speedup vs baseline: 1.3493x; 1.3493x over previous
import jax
import jax.numpy as jnp
from jax.experimental import pallas as pl

_N = 512
_L = 24
_B = 64
_C = 24
_TB = 64


def _prologue_body(adj_ref, ws1_ref, wn1_ref, ws2_ref, wn2_ref, b1c_ref,
                   b2c_ref, at_ref, a2t_ref, adj16_ref, bv_ref, mcat_ref):
    adj = adj_ref[...]
    mask = (adj != 0.0).astype(jnp.float32)
    deg = jnp.sum(mask, axis=0, keepdims=True)
    inv = jnp.where(deg > 0.0, 1.0 / jnp.maximum(deg, 1.0), 0.0)
    at = mask * inv
    at_ref[...] = at.astype(jnp.bfloat16)
    a2t_ref[...] = jnp.dot(at, at,
                           preferred_element_type=jnp.float32).astype(jnp.bfloat16)
    adj16_ref[...] = adj.astype(jnp.bfloat16)
    ws1 = ws1_ref[...]
    wn1 = wn1_ref[...]
    ws2 = ws2_ref[...]
    wn2 = wn2_ref[...]
    mcat_ref[0:24, :] = jnp.dot(ws2, ws1, preferred_element_type=jnp.float32)
    mcat_ref[24:48, :] = (jnp.dot(ws2, wn1, preferred_element_type=jnp.float32)
                          + jnp.dot(wn2, ws1, preferred_element_type=jnp.float32))
    mcat_ref[48:72, :] = jnp.dot(wn2, wn1, preferred_element_type=jnp.float32)
    b1c = b1c_ref[...]
    bconst = jnp.dot(ws2, b1c, preferred_element_type=jnp.float32) + b2c_ref[...]
    bneigh = jnp.dot(wn2, b1c, preferred_element_type=jnp.float32)
    ind = (deg > 0.0).astype(jnp.float32)
    bv_ref[...] = (jnp.tile(bconst, (_B, 1))
                   + jnp.tile(bneigh, (_B, 1)) * ind)


def _main_body(x_ref, at_ref, a2t_ref, adj_ref, bv_ref, mcat_ref, out_ref):
    xv = x_ref[...].reshape(_TB, _L, _N)
    mcat_b = jnp.broadcast_to(mcat_ref[...][None], (_TB, 3 * _L, _L))
    xall = jax.lax.dot_general(
        mcat_b, xv, (((2,), (1,)), ((0,), (0,))),
        preferred_element_type=jnp.float32)
    x1 = xall[:, 0:_L, :].reshape(_TB * _L, _N)
    x2 = xall[:, _L:2 * _L, :].reshape(_TB * _L, _N).astype(jnp.bfloat16)
    x3 = xall[:, 2 * _L:3 * _L, :].reshape(_TB * _L, _N).astype(jnp.bfloat16)
    res = (x1
           + jnp.dot(x2, at_ref[...], preferred_element_type=jnp.float32)
           + jnp.dot(x3, a2t_ref[...], preferred_element_type=jnp.float32)
           + bv_ref[...])
    z = jnp.dot(res.astype(jnp.bfloat16), adj_ref[...],
                preferred_element_type=jnp.float32)
    out_ref[:, 0, :, :] = (3.5 * res).reshape(_TB, _L, _N)
    out_ref[:, 1, :, :] = z.reshape(_TB, _L, _N)


@jax.jit
def kernel(x, adj, W_self1, W_neigh1, b1, W_self2, W_neigh2, b2):
    B, C, N, L = x.shape

    at, a2t, adj16, bv, mcat = pl.pallas_call(
        _prologue_body,
        out_shape=(
            jax.ShapeDtypeStruct((N, N), jnp.bfloat16),
            jax.ShapeDtypeStruct((N, N), jnp.bfloat16),
            jax.ShapeDtypeStruct((N, N), jnp.bfloat16),
            jax.ShapeDtypeStruct((B * L, N), jnp.float32),
            jax.ShapeDtypeStruct((3 * L, L), jnp.float32),
        ),
    )(adj, W_self1, W_neigh1, W_self2, W_neigh2, b1[:, None], b2[:, None])

    xt = jnp.swapaxes(x, 2, 3)

    outt = pl.pallas_call(
        _main_body,
        grid=(C,),
        in_specs=[
            pl.BlockSpec((_TB, 1, L, N), lambda c: (0, c, 0, 0)),
            pl.BlockSpec((N, N), lambda c: (0, 0)),
            pl.BlockSpec((N, N), lambda c: (0, 0)),
            pl.BlockSpec((N, N), lambda c: (0, 0)),
            pl.BlockSpec((_TB * L, N), lambda c: (0, 0)),
            pl.BlockSpec((3 * L, L), lambda c: (0, 0)),
        ],
        out_specs=pl.BlockSpec((_TB, 2, L, N), lambda c: (0, c, 0, 0)),
        out_shape=jax.ShapeDtypeStruct((B, 2 * C, L, N), jnp.float32),
    )(xt, at, a2t, adj16, bv, mcat)

    return jnp.swapaxes(outt, 2, 3)


# speedup vs baseline: 1.3658x; 1.0122x over previous
import jax
import jax.numpy as jnp
from jax.experimental import pallas as pl

_N = 512
_L = 24
_B = 64
_C = 24
_TC = 2


def _prologue_body(adj_ref, ws1_ref, wn1_ref, ws2_ref, wn2_ref, b1c_ref,
                   b2c_ref, at_ref, a2t_ref, bv_ref, mcat_ref):
    adj = adj_ref[...]
    mask = (adj != 0.0).astype(jnp.float32)
    deg = jnp.sum(mask, axis=0, keepdims=True)
    inv = jnp.where(deg > 0.0, 1.0 / jnp.maximum(deg, 1.0), 0.0)
    at = mask * inv
    at_ref[...] = at
    a2t_ref[...] = jnp.dot(at, at, preferred_element_type=jnp.float32)
    ws1 = ws1_ref[...]
    wn1 = wn1_ref[...]
    ws2 = ws2_ref[...]
    wn2 = wn2_ref[...]
    mcat_ref[0:24, :] = jnp.dot(ws2, ws1, preferred_element_type=jnp.float32)
    mcat_ref[24:48, :] = (jnp.dot(ws2, wn1, preferred_element_type=jnp.float32)
                          + jnp.dot(wn2, ws1, preferred_element_type=jnp.float32))
    mcat_ref[48:72, :] = jnp.dot(wn2, wn1, preferred_element_type=jnp.float32)
    b1c = b1c_ref[...]
    bconst = jnp.dot(ws2, b1c, preferred_element_type=jnp.float32) + b2c_ref[...]
    bneigh = jnp.dot(wn2, b1c, preferred_element_type=jnp.float32)
    ind = (deg > 0.0).astype(jnp.float32)
    bv_ref[...] = (jnp.tile(bconst, (_B, 1))
                   + jnp.tile(bneigh, (_B, 1)) * ind)


def _main_body(x_ref, at_ref, a2t_ref, adj_ref, bv_ref, mcat_ref, out_ref):
    nb = _B * _TC
    xv = x_ref[...].reshape(nb, _L, _N)
    mcat_b = jnp.broadcast_to(mcat_ref[...][None], (nb, 3 * _L, _L))
    xall = jax.lax.dot_general(
        mcat_b, xv, (((2,), (1,)), ((0,), (0,))),
        preferred_element_type=jnp.float32)
    x1 = xall[:, 0:_L, :].reshape(nb * _L, _N)
    x2 = xall[:, _L:2 * _L, :].reshape(nb * _L, _N)
    x3 = xall[:, 2 * _L:3 * _L, :].reshape(nb * _L, _N)
    resnb = (x1
             + jnp.dot(x2, at_ref[...], preferred_element_type=jnp.float32)
             + jnp.dot(x3, a2t_ref[...], preferred_element_type=jnp.float32)
             ).reshape(_B, _TC, _L, _N)
    bvv = bv_ref[...].reshape(_B, _L, _N)
    for j in range(_TC):
        r = resnb[:, j, :, :] + bvv
        z = jnp.dot(r.reshape(_B * _L, _N), adj_ref[...],
                    preferred_element_type=jnp.float32)
        out_ref[:, 2 * j, :, :] = 3.5 * r
        out_ref[:, 2 * j + 1, :, :] = z.reshape(_B, _L, _N)


@jax.jit
def kernel(x, adj, W_self1, W_neigh1, b1, W_self2, W_neigh2, b2):
    B, C, N, L = x.shape

    at, a2t, bv, mcat = pl.pallas_call(
        _prologue_body,
        out_shape=(
            jax.ShapeDtypeStruct((N, N), jnp.float32),
            jax.ShapeDtypeStruct((N, N), jnp.float32),
            jax.ShapeDtypeStruct((B * L, N), jnp.float32),
            jax.ShapeDtypeStruct((3 * L, L), jnp.float32),
        ),
    )(adj, W_self1, W_neigh1, W_self2, W_neigh2, b1[:, None], b2[:, None])

    xt = jnp.swapaxes(x, 2, 3)

    outt = pl.pallas_call(
        _main_body,
        grid=(C // _TC,),
        in_specs=[
            pl.BlockSpec((B, _TC, L, N), lambda c: (0, c, 0, 0)),
            pl.BlockSpec((N, N), lambda c: (0, 0)),
            pl.BlockSpec((N, N), lambda c: (0, 0)),
            pl.BlockSpec((N, N), lambda c: (0, 0)),
            pl.BlockSpec((B * L, N), lambda c: (0, 0)),
            pl.BlockSpec((3 * L, L), lambda c: (0, 0)),
        ],
        out_specs=pl.BlockSpec((B, 2 * _TC, L, N), lambda c: (0, c, 0, 0)),
        out_shape=jax.ShapeDtypeStruct((B, 2 * C, L, N), jnp.float32),
    )(xt, at, a2t, adj, bv, mcat)

    return jnp.swapaxes(outt, 2, 3)


# speedup vs baseline: 1.3904x; 1.0180x over previous
import jax
import jax.numpy as jnp
from jax.experimental import pallas as pl

_N = 512
_L = 24
_B = 64
_C = 24
_TC = 2


def _prologue_body(adj_ref, ws1_ref, wn1_ref, ws2_ref, wn2_ref, b1c_ref,
                   b2c_ref, at_ref, a2t_ref, bv_ref, zb_ref, mcat_ref):
    adj = adj_ref[...]
    mask = (adj != 0.0).astype(jnp.float32)
    deg = jnp.sum(mask, axis=0, keepdims=True)
    inv = jnp.where(deg > 0.0, 1.0 / jnp.maximum(deg, 1.0), 0.0)
    at = mask * inv
    at_ref[...] = at
    a2t_ref[...] = jnp.dot(at, at, preferred_element_type=jnp.float32)
    ws1 = ws1_ref[...]
    wn1 = wn1_ref[...]
    ws2 = ws2_ref[...]
    wn2 = wn2_ref[...]
    mcat_ref[0:24, :] = jnp.dot(ws2, ws1, preferred_element_type=jnp.float32)
    mcat_ref[24:48, :] = (jnp.dot(ws2, wn1, preferred_element_type=jnp.float32)
                          + jnp.dot(wn2, ws1, preferred_element_type=jnp.float32))
    mcat_ref[48:72, :] = jnp.dot(wn2, wn1, preferred_element_type=jnp.float32)
    b1c = b1c_ref[...]
    bconst = jnp.dot(ws2, b1c, preferred_element_type=jnp.float32) + b2c_ref[...]
    bneigh = jnp.dot(wn2, b1c, preferred_element_type=jnp.float32)
    ind = (deg > 0.0).astype(jnp.float32)
    bm = bconst + bneigh * ind
    bv_ref[...] = bm
    zb_ref[...] = jnp.dot(bm, adj, preferred_element_type=jnp.float32)


def _main_body(x_ref, at_ref, a2t_ref, adj_ref, bv_ref, zb_ref, mcat_ref,
               out_ref):
    nb = _B * _TC
    xv = x_ref[...].reshape(nb, _L, _N)
    mcat_b = jnp.broadcast_to(mcat_ref[...][None], (nb, 3 * _L, _L))
    xall = jax.lax.dot_general(
        mcat_b, xv, (((2,), (1,)), ((0,), (0,))),
        preferred_element_type=jnp.float32)
    x1 = xall[:, 0:_L, :].reshape(nb * _L, _N)
    x2 = xall[:, _L:2 * _L, :].reshape(nb * _L, _N)
    x3 = xall[:, 2 * _L:3 * _L, :].reshape(nb * _L, _N)
    resnb = (x1
             + jnp.dot(x2, at_ref[...], preferred_element_type=jnp.float32)
             + jnp.dot(x3, a2t_ref[...], preferred_element_type=jnp.float32))
    znb = jnp.dot(resnb, adj_ref[...],
                  preferred_element_type=jnp.float32)
    res4 = resnb.reshape(_B, _TC, _L, _N)
    z4 = znb.reshape(_B, _TC, _L, _N)
    bvv = bv_ref[...][None]
    zbv = zb_ref[...][None]
    for j in range(_TC):
        out_ref[:, 2 * j, :, :] = 3.5 * (res4[:, j, :, :] + bvv)
        out_ref[:, 2 * j + 1, :, :] = z4[:, j, :, :] + zbv


@jax.jit
def kernel(x, adj, W_self1, W_neigh1, b1, W_self2, W_neigh2, b2):
    B, C, N, L = x.shape

    at, a2t, bv, zb, mcat = pl.pallas_call(
        _prologue_body,
        out_shape=(
            jax.ShapeDtypeStruct((N, N), jnp.float32),
            jax.ShapeDtypeStruct((N, N), jnp.float32),
            jax.ShapeDtypeStruct((L, N), jnp.float32),
            jax.ShapeDtypeStruct((L, N), jnp.float32),
            jax.ShapeDtypeStruct((3 * L, L), jnp.float32),
        ),
    )(adj, W_self1, W_neigh1, W_self2, W_neigh2, b1[:, None], b2[:, None])

    xt = jnp.swapaxes(x, 2, 3)

    outt = pl.pallas_call(
        _main_body,
        grid=(C // _TC,),
        in_specs=[
            pl.BlockSpec((B, _TC, L, N), lambda c: (0, c, 0, 0)),
            pl.BlockSpec((N, N), lambda c: (0, 0)),
            pl.BlockSpec((N, N), lambda c: (0, 0)),
            pl.BlockSpec((N, N), lambda c: (0, 0)),
            pl.BlockSpec((L, N), lambda c: (0, 0)),
            pl.BlockSpec((L, N), lambda c: (0, 0)),
            pl.BlockSpec((3 * L, L), lambda c: (0, 0)),
        ],
        out_specs=pl.BlockSpec((B, 2 * _TC, L, N), lambda c: (0, c, 0, 0)),
        out_shape=jax.ShapeDtypeStruct((B, 2 * C, L, N), jnp.float32),
    )(xt, at, a2t, adj, bv, zb, mcat)

    return jnp.swapaxes(outt, 2, 3)
